# baseline (speedup 1.0000x reference)
import functools

import jax
import jax.numpy as jnp
from jax import lax
from jax.experimental import pallas as pl
from jax.experimental.pallas import tpu as pltpu
from jax.experimental.pallas import tpu_sc as plsc

INV_QUAN = 50.0
NHEAD = 16
LANES = 16
NC, NS = 2, 16
NW = NC * NS


@functools.lru_cache(maxsize=None)
def _rpe_sc_kernel(max_len, n_points):
    ppw = n_points // NW
    SLAB = 1024
    n_slabs = ppw // SLAB
    g_per_slab = SLAB // LANES
    tbl_words = max_len * NHEAD

    mesh = plsc.VectorSubcoreMesh(core_axis_name="c", subcore_axis_name="s")

    @functools.partial(
        pl.kernel,
        mesh=mesh,
        out_type=jax.ShapeDtypeStruct((n_points * NHEAD,), jnp.float32),
        compiler_params=pltpu.CompilerParams(needs_layout_passes=False),
        scratch_types=[
            pltpu.VMEM((tbl_words,), jnp.float32),
            pltpu.VMEM((ppw * 3,), jnp.float32),
            pltpu.VMEM((SLAB * NHEAD,), jnp.float32),
            pltpu.VMEM((SLAB * NHEAD,), jnp.float32),
            pltpu.SemaphoreType.DMA,
            pltpu.SemaphoreType.DMA,
        ],
    )
    def k(crd_hbm, table_hbm, out_hbm, tbl_v, crd_v, ob0_v, ob1_v, sem0, sem1):
        wid = lax.axis_index("s") * NC + lax.axis_index("c")
        base = wid * ppw
        tbl_cp = pltpu.async_copy(table_hbm, tbl_v, sem0)
        pltpu.sync_copy(crd_hbm.at[pl.ds(base * 3, ppw * 3)], crd_v)
        tbl_cp.wait()

        lanes = lax.iota(jnp.int32, LANES)
        lanes16 = lanes * NHEAD

        def group(out_v, slab, g):
            p3 = (slab * SLAB + g * LANES + lanes) * 3
            x = plsc.load_gather(crd_v, [p3])
            y = plsc.load_gather(crd_v, [p3 + 1])
            z = plsc.load_gather(crd_v, [p3 + 2])
            s = jnp.maximum(x * x + y * y + z * z, 1e-30)
            bits = lax.bitcast_convert_type(s, jnp.int32)
            r = lax.bitcast_convert_type(0x5F3759DF - (bits >> 1), jnp.float32)
            hs = 0.5 * s
            r = r * (1.5 - hs * r * r)
            r = r * (1.5 - hs * r * r)
            r = r * (1.5 - hs * r * r)
            d = s * r * INV_QUAN
            i1 = d.astype(jnp.int32)
            i2 = i1 + 1
            w1 = i2.astype(jnp.float32) - d
            w2 = d - i1.astype(jnp.float32)
            b1 = jnp.minimum(i1, max_len - 1) * NHEAD
            b2 = jnp.minimum(i2, max_len - 1) * NHEAD
            ob = g * (LANES * NHEAD) + lanes16
            for c in range(NHEAD):
                e1 = plsc.load_gather(tbl_v, [b1 + c])
                e2 = plsc.load_gather(tbl_v, [b2 + c])
                plsc.store_scatter(out_v, [ob + c], e1 * w1 + e2 * w2)

        bufs = (ob0_v, ob1_v)
        sems = (sem0, sem1)
        out_cps = [None, None]
        for slab in range(n_slabs):
            i = slab % 2
            if out_cps[i] is not None:
                out_cps[i].wait()
            plsc.parallel_loop(0, g_per_slab, unroll=1)(
                functools.partial(group, bufs[i], slab)
            )
            out_cps[i] = pltpu.async_copy(
                bufs[i],
                out_hbm.at[pl.ds((base + slab * SLAB) * NHEAD, SLAB * NHEAD)],
                sems[i],
            )
        for cp in out_cps:
            if cp is not None:
                cp.wait()

    return k


def kernel(batch_rel_coords, pos_embed):
    b, p, _ = batch_rel_coords.shape
    n = b * p
    max_len = pos_embed.shape[0]
    out = _rpe_sc_kernel(max_len, n)(
        batch_rel_coords.reshape(-1), pos_embed.reshape(-1)
    )
    return out.reshape(b, p, NHEAD)
